# baseline (speedup 1.0000x reference)
import functools

import jax
import jax.numpy as jnp
from jax import lax
from jax.experimental import pallas as pl
from jax.experimental.pallas import tpu as pltpu
from jax.experimental.pallas import tpu_sc as plsc

PROMPT_LENGTH = 2
CTX_LEN = 77
BATCH = 4096
VOCAB = 49408
D_TEXT = 512

NUM_CORES = 2
NUM_SUBCORES = 16
NUM_WORKERS = NUM_CORES * NUM_SUBCORES
B_PER_WORKER = BATCH // NUM_WORKERS
HALF_STEPS = B_PER_WORKER // 2

CTX_PAD = 80
MAIN_ROWS = 72
TAIL_BASE = 72
TAIL_ROWS = 5

LANES = 16
VECS_PER_ROW = D_TEXT // LANES


def _build_gather():
    mesh = plsc.VectorSubcoreMesh(
        core_axis_name="c",
        subcore_axis_name="s",
        num_cores=NUM_CORES,
        num_subcores=NUM_SUBCORES,
    )

    @functools.partial(
        pl.kernel,
        mesh=mesh,
        out_type=jax.ShapeDtypeStruct((BATCH, CTX_PAD, D_TEXT), jnp.float32),
        scratch_types=[
            pltpu.VMEM((B_PER_WORKER, CTX_PAD), jnp.int32),
            pltpu.VMEM((CTX_PAD, D_TEXT), jnp.float32),
            pltpu.VMEM((CTX_PAD, D_TEXT), jnp.float32),
            pltpu.VMEM((TAIL_ROWS, D_TEXT), jnp.float32),
            pltpu.VMEM((TAIL_ROWS, D_TEXT), jnp.float32),
            pltpu.VMEM((PROMPT_LENGTH, D_TEXT), jnp.float32),
            pltpu.SemaphoreType.DMA,
            pltpu.SemaphoreType.DMA,
            pltpu.SemaphoreType.DMA,
            pltpu.SemaphoreType.DMA,
        ],
    )
    def gather_kernel(table_hbm, idx_hbm, ctx_hbm, out_hbm,
                      idx_v, rows0, rows1, tail0, tail1, ctx_v,
                      gsem0, gsem1, ssem0, ssem1):
        wid = lax.axis_index("s") * NUM_CORES + lax.axis_index("c")
        b0 = wid * B_PER_WORKER
        pltpu.sync_copy(idx_hbm.at[pl.ds(b0, B_PER_WORKER)], idx_v)
        pltpu.sync_copy(ctx_hbm, ctx_v)

        def g_issue(i, rv, gs):
            pltpu.async_copy(table_hbm.at[idx_v.at[i]], rv, gs)

        def g_wait(rv, gs):
            pltpu.make_async_copy(table_hbm.at[pl.ds(0, CTX_PAD)], rv, gs).wait()

        def s_issue(i, rv, tv, ss):
            pltpu.async_copy(rv, out_hbm.at[b0 + i], ss)

        def s_wait(rv, tv, ss):
            pltpu.make_async_copy(rv, out_hbm.at[b0], ss).wait()

        def patch(rv, tv):
            pass

        def finish(i, rv, tv, gs, ss):
            g_wait(rv, gs)
            patch(rv, tv)
            s_issue(i, rv, tv, ss)

        g_issue(0, rows0, gsem0)
        finish(0, rows0, tail0, gsem0, ssem0)
        g_issue(1, rows1, gsem1)
        finish(1, rows1, tail1, gsem1, ssem1)
        s_wait(rows0, tail0, ssem0)
        g_issue(2, rows0, gsem0)

        def body(c2, carry):
            i0 = 2 * c2
            i1 = i0 + 1
            finish(i0, rows0, tail0, gsem0, ssem0)
            s_wait(rows1, tail1, ssem1)
            g_issue(i1, rows1, gsem1)
            finish(i1, rows1, tail1, gsem1, ssem1)
            s_wait(rows0, tail0, ssem0)
            g_issue(i0 + 2, rows0, gsem0)
            return carry

        lax.fori_loop(1, HALF_STEPS - 1, body, 0)

        i0 = 2 * (HALF_STEPS - 1)
        finish(i0, rows0, tail0, gsem0, ssem0)
        s_wait(rows1, tail1, ssem1)
        g_issue(i0 + 1, rows1, gsem1)
        finish(i0 + 1, rows1, tail1, gsem1, ssem1)
        s_wait(rows0, tail0, ssem0)
        s_wait(rows1, tail1, ssem1)

    return gather_kernel


_gather = _build_gather()


@jax.jit
def kernel(fmri, token, token_embedding, ctx_text, ctx_img, text_prompts, img_prompts):
    idx = token[:, 0, :].astype(jnp.int32)
    idx = jnp.pad(idx, ((0, 0), (0, CTX_PAD - CTX_LEN)))
    texts = _gather(token_embedding, idx, ctx_text)[:, :CTX_LEN, :]
    return (fmri, texts, ctx_img, text_prompts, img_prompts)


# speedup vs baseline: 1.9704x; 1.9704x over previous
import functools

import jax
import jax.numpy as jnp
from jax import lax
from jax.experimental import pallas as pl
from jax.experimental.pallas import tpu as pltpu
from jax.experimental.pallas import tpu_sc as plsc

PROMPT_LENGTH = 2
CTX_LEN = 77
BATCH = 4096
VOCAB = 49408
D_TEXT = 512

NUM_CORES = 2
NUM_SUBCORES = 16
NUM_WORKERS = NUM_CORES * NUM_SUBCORES
B_PER_WORKER = BATCH // NUM_WORKERS
HALF_STEPS = B_PER_WORKER // 2

CTX_PAD = 80
MAIN_ROWS = 72
TAIL_BASE = 72
TAIL_ROWS = 5

LANES = 16
VECS_PER_ROW = D_TEXT // LANES


def _build_gather():
    mesh = plsc.VectorSubcoreMesh(
        core_axis_name="c",
        subcore_axis_name="s",
        num_cores=NUM_CORES,
        num_subcores=NUM_SUBCORES,
    )

    @functools.partial(
        pl.kernel,
        mesh=mesh,
        out_type=jax.ShapeDtypeStruct((BATCH, CTX_PAD, D_TEXT), jnp.float32),
        scratch_types=[
            pltpu.VMEM((B_PER_WORKER, CTX_PAD), jnp.int32),
            pltpu.VMEM((CTX_PAD, D_TEXT), jnp.float32),
            pltpu.VMEM((CTX_PAD, D_TEXT), jnp.float32),
            pltpu.VMEM((TAIL_ROWS, D_TEXT), jnp.float32),
            pltpu.VMEM((TAIL_ROWS, D_TEXT), jnp.float32),
            pltpu.VMEM((PROMPT_LENGTH, D_TEXT), jnp.float32),
            pltpu.SemaphoreType.DMA,
            pltpu.SemaphoreType.DMA,
            pltpu.SemaphoreType.DMA,
            pltpu.SemaphoreType.DMA,
        ],
    )
    def gather_kernel(table_hbm, idx_hbm, ctx_hbm, out_hbm,
                      idx_v, rows0, rows1, tail0, tail1, ctx_v,
                      gsem0, gsem1, ssem0, ssem1):
        wid = lax.axis_index("s") * NUM_CORES + lax.axis_index("c")
        b0 = wid * B_PER_WORKER
        pltpu.sync_copy(idx_hbm.at[pl.ds(b0, B_PER_WORKER)], idx_v)
        pltpu.sync_copy(ctx_hbm, ctx_v)

        def g_issue(i, rv, gs):
            pltpu.async_copy(table_hbm.at[idx_v.at[i]], rv, gs)

        def g_wait(rv, gs):
            pltpu.make_async_copy(table_hbm.at[pl.ds(0, CTX_PAD)], rv, gs).wait()

        def s_issue(i, rv, tv, ss):
            pltpu.async_copy(rv, out_hbm.at[b0 + i], ss)

        def s_wait(rv, tv, ss):
            pltpu.make_async_copy(rv, out_hbm.at[b0], ss).wait()

        def patch(rv, tv):
            pass

        def finish(i, rv, tv, gs, ss):
            g_wait(rv, gs)
            patch(rv, tv)
            s_issue(i, rv, tv, ss)

        g_issue(0, rows0, gsem0)
        finish(0, rows0, tail0, gsem0, ssem0)
        g_issue(1, rows1, gsem1)
        finish(1, rows1, tail1, gsem1, ssem1)
        s_wait(rows0, tail0, ssem0)
        g_issue(2, rows0, gsem0)

        def body(c2, carry):
            i0 = 2 * c2
            i1 = i0 + 1
            finish(i0, rows0, tail0, gsem0, ssem0)
            s_wait(rows1, tail1, ssem1)
            g_issue(i1, rows1, gsem1)
            finish(i1, rows1, tail1, gsem1, ssem1)
            s_wait(rows0, tail0, ssem0)
            g_issue(i0 + 2, rows0, gsem0)
            return carry

        lax.fori_loop(1, HALF_STEPS - 1, body, 0)

        i0 = 2 * (HALF_STEPS - 1)
        finish(i0, rows0, tail0, gsem0, ssem0)
        s_wait(rows1, tail1, ssem1)
        g_issue(i0 + 1, rows1, gsem1)
        finish(i0 + 1, rows1, tail1, gsem1, ssem1)
        s_wait(rows0, tail0, ssem0)
        s_wait(rows1, tail1, ssem1)

    return gather_kernel


_gather = _build_gather()


@jax.jit
def kernel(fmri, token, token_embedding, ctx_text, ctx_img, text_prompts, img_prompts):
    idx = token[:, 0, :].astype(jnp.int32)
    idx = jnp.concatenate([idx, idx[:, -(CTX_PAD - CTX_LEN):]], axis=1)
    texts = _gather(token_embedding, idx, ctx_text)[:, :CTX_LEN, :]
    return (fmri, texts, ctx_img, text_prompts, img_prompts)


# speedup vs baseline: 2.0350x; 1.0328x over previous
import functools

import jax
import jax.numpy as jnp
from jax import lax
from jax.experimental import pallas as pl
from jax.experimental.pallas import tpu as pltpu
from jax.experimental.pallas import tpu_sc as plsc

PROMPT_LENGTH = 2
CTX_LEN = 77
BATCH = 4096
VOCAB = 49408
D_TEXT = 512

NUM_CORES = 2
NUM_SUBCORES = 16
NUM_WORKERS = NUM_CORES * NUM_SUBCORES
B_PER_WORKER = BATCH // NUM_WORKERS
HALF_STEPS = B_PER_WORKER // 2

CTX_PAD = 80
MAIN_ROWS = 72
TAIL_BASE = 72
TAIL_ROWS = 5

LANES = 16
VECS_PER_ROW = D_TEXT // LANES


def _build_gather():
    mesh = plsc.VectorSubcoreMesh(
        core_axis_name="c",
        subcore_axis_name="s",
        num_cores=NUM_CORES,
        num_subcores=NUM_SUBCORES,
    )

    @functools.partial(
        pl.kernel,
        mesh=mesh,
        out_type=jax.ShapeDtypeStruct((BATCH, CTX_LEN, D_TEXT), jnp.float32),
        scratch_types=[
            pltpu.VMEM((B_PER_WORKER, CTX_PAD), jnp.int32),
            pltpu.VMEM((CTX_PAD, D_TEXT), jnp.float32),
            pltpu.VMEM((CTX_PAD, D_TEXT), jnp.float32),
            pltpu.VMEM((TAIL_ROWS, D_TEXT), jnp.float32),
            pltpu.VMEM((TAIL_ROWS, D_TEXT), jnp.float32),
            pltpu.VMEM((PROMPT_LENGTH, D_TEXT), jnp.float32),
            pltpu.SemaphoreType.DMA,
            pltpu.SemaphoreType.DMA,
            pltpu.SemaphoreType.DMA,
            pltpu.SemaphoreType.DMA,
        ],
    )
    def gather_kernel(table_hbm, idx_hbm, ctx_hbm, out_hbm,
                      idx_v, rows0, rows1, tail0, tail1, ctx_v,
                      gsem0, gsem1, ssem0, ssem1):
        wid = lax.axis_index("s") * NUM_CORES + lax.axis_index("c")
        b0 = wid * B_PER_WORKER
        pltpu.sync_copy(idx_hbm.at[pl.ds(b0, B_PER_WORKER)], idx_v)
        pltpu.sync_copy(ctx_hbm, ctx_v)

        def g_issue(i, rv, gs):
            pltpu.async_copy(table_hbm.at[idx_v.at[i]], rv, gs)

        def g_wait(rv, gs):
            pltpu.make_async_copy(table_hbm.at[pl.ds(0, CTX_PAD)], rv, gs).wait()

        def s_issue(i, rv, tv, ss):
            pltpu.async_copy(rv.at[pl.ds(0, MAIN_ROWS)],
                             out_hbm.at[b0 + i, pl.ds(0, MAIN_ROWS)], ss)
            pltpu.async_copy(tv, out_hbm.at[b0 + i, pl.ds(TAIL_BASE, TAIL_ROWS)], ss)

        def s_wait(rv, tv, ss):
            pltpu.make_async_copy(rv.at[pl.ds(0, MAIN_ROWS)],
                                  out_hbm.at[b0, pl.ds(0, MAIN_ROWS)], ss).wait()
            pltpu.make_async_copy(tv, out_hbm.at[b0, pl.ds(TAIL_BASE, TAIL_ROWS)],
                                  ss).wait()

        def patch(rv, tv):
            for p in range(PROMPT_LENGTH):
                for k in range(VECS_PER_ROW):
                    sl = pl.ds(k * LANES, LANES)
                    rv[1 + p, sl] = ctx_v[p, sl]
            for t in range(TAIL_ROWS):
                for k in range(VECS_PER_ROW):
                    sl = pl.ds(k * LANES, LANES)
                    tv[t, sl] = rv[TAIL_BASE + t, sl]

        def finish(i, rv, tv, gs, ss):
            g_wait(rv, gs)
            patch(rv, tv)
            s_issue(i, rv, tv, ss)

        g_issue(0, rows0, gsem0)
        finish(0, rows0, tail0, gsem0, ssem0)
        g_issue(1, rows1, gsem1)
        finish(1, rows1, tail1, gsem1, ssem1)
        s_wait(rows0, tail0, ssem0)
        g_issue(2, rows0, gsem0)

        def body(c2, carry):
            i0 = 2 * c2
            i1 = i0 + 1
            finish(i0, rows0, tail0, gsem0, ssem0)
            s_wait(rows1, tail1, ssem1)
            g_issue(i1, rows1, gsem1)
            finish(i1, rows1, tail1, gsem1, ssem1)
            s_wait(rows0, tail0, ssem0)
            g_issue(i0 + 2, rows0, gsem0)
            return carry

        lax.fori_loop(1, HALF_STEPS - 1, body, 0)

        i0 = 2 * (HALF_STEPS - 1)
        finish(i0, rows0, tail0, gsem0, ssem0)
        s_wait(rows1, tail1, ssem1)
        g_issue(i0 + 1, rows1, gsem1)
        finish(i0 + 1, rows1, tail1, gsem1, ssem1)
        s_wait(rows0, tail0, ssem0)
        s_wait(rows1, tail1, ssem1)

    return gather_kernel


_gather = _build_gather()


@jax.jit
def kernel(fmri, token, token_embedding, ctx_text, ctx_img, text_prompts, img_prompts):
    idx = token[:, 0, :].astype(jnp.int32)
    idx = jnp.concatenate([idx, idx[:, -(CTX_PAD - CTX_LEN):]], axis=1)
    texts = _gather(token_embedding, idx, ctx_text)
    return (fmri, texts, ctx_img, text_prompts, img_prompts)
